# Initial kernel scaffold; baseline (speedup 1.0000x reference)
import jax
import jax.numpy as jnp
from jax.experimental import pallas as pl


def kernel(x, zone, region_adjacency, S, Wq, bq, Wk, bk, Wv, bv, Wo, bo, Wg, bg, prelu_a):
    raise NotImplementedError("write your pallas kernel here")


# speedup vs baseline: 11.1522x; 11.1522x over previous
import jax
import jax.numpy as jnp
from jax.experimental import pallas as pl

N = 50000
H = 256
NH = 4
DS = H // NH
R = 1000
E = 16000
NQ = N // NH
BJ = 512
NB = 25
NQP = BJ * NB
EC = 1000
NEC = E // EC

_FMAX = 3.4028234663852886e38


def _attn_body(x_ref, zone_ref, s_ref, wq_ref, bq_ref, wk_ref, bk_ref,
               wv_ref, bv_ref, num_ref, den_ref):
    i = pl.program_id(0)

    @pl.when(i == 0)
    def _():
        num_ref[...] = jnp.zeros_like(num_ref)
        den_ref[...] = jnp.zeros_like(den_ref)

    qseed = jnp.dot(s_ref[...], wq_ref[...],
                    preferred_element_type=jnp.float32) + bq_ref[...]

    gi = jax.lax.broadcasted_iota(jnp.int32, (H, NH), 0) // DS
    gj = jax.lax.broadcasted_iota(jnp.int32, (H, NH), 1)
    G = (gi == gj).astype(jnp.float32)

    vws = []
    exs = []
    for h in range(NH):
        xh = x_ref[h]
        kh = jnp.dot(xh, wk_ref[...],
                     preferred_element_type=jnp.float32) + bk_ref[...]
        vh = jnp.dot(xh, wv_ref[...],
                     preferred_element_type=jnp.float32) + bv_ref[...]
        qtile = jnp.tile(qseed[:, h * DS:(h + 1) * DS], (1, NH))
        ph = jnp.dot(kh * qtile, G,
                     preferred_element_type=jnp.float32)
        exh = jnp.exp(ph * (1.0 / 16.0))
        exs.append(exh)
        vws.append([vh[:, g * DS:(g + 1) * DS] * exh[:, g:g + 1]
                    for g in range(NH)])

    iota_r = jax.lax.broadcasted_iota(jnp.int32, (R, BJ), 0)
    zall = zone_ref[0]
    accs = [jnp.zeros((R, DS), jnp.float32) for _ in range(NH)]
    dacc = jnp.zeros((R, NH), jnp.float32)
    for g in range(NH):
        oh = (iota_r == zall[g:g + 1, :]).astype(jnp.float32)
        exg = jnp.concatenate([exs[h][:, g:g + 1] for h in range(NH)],
                              axis=1)
        dacc = dacc + jnp.dot(oh, exg, preferred_element_type=jnp.float32)
        for h in range(NH):
            accs[h] = accs[h] + jnp.dot(oh, vws[h][g],
                                        preferred_element_type=jnp.float32)
    num_ref[...] += jnp.concatenate(accs, axis=1)
    den_ref[...] += dacc


def _head_body(num_ref, den_ref, s_ref, wq_ref, bq_ref, wo_ref, bo_ref,
               wg_ref, bg_ref, adj_ref, pa_ref, out_ref):
    qseed = jnp.dot(s_ref[...], wq_ref[...],
                    preferred_element_type=jnp.float32) + bq_ref[...]

    gi = jax.lax.broadcasted_iota(jnp.int32, (NH, H), 1) // DS
    gj = jax.lax.broadcasted_iota(jnp.int32, (NH, H), 0)
    GT = (gi == gj).astype(jnp.float32)
    denR = jnp.dot(den_ref[...], GT, preferred_element_type=jnp.float32)

    region = num_ref[...] / (denR + 1e-16)
    O = qseed + region
    O = O + jax.nn.relu(jnp.dot(O, wo_ref[...],
                                preferred_element_type=jnp.float32) + bo_ref[...])
    hW = jnp.dot(O, wg_ref[...], preferred_element_type=jnp.float32)

    iota_re = jax.lax.broadcasted_iota(jnp.int32, (R, EC), 0)

    def _edge_chunk(c, M):
        rows = adj_ref[c, 0:1, :]
        cols = adj_ref[c, 1:2, :]
        ohr = (iota_re == rows).astype(jnp.bfloat16)
        ohc = (iota_re == cols).astype(jnp.bfloat16)
        return M + jax.lax.dot_general(
            ohc, ohr, (((1,), (1,)), ((), ())),
            preferred_element_type=jnp.float32)

    M = jax.lax.fori_loop(0, NEC, _edge_chunk,
                          jnp.zeros((R, R), jnp.float32))
    ri = jax.lax.broadcasted_iota(jnp.int32, (R, R), 0)
    rj = jax.lax.broadcasted_iota(jnp.int32, (R, R), 1)
    M = M + (ri == rj).astype(jnp.float32)

    deg = jnp.sum(M, axis=1, keepdims=True)
    dinv = jax.lax.rsqrt(deg)
    hfin = dinv * jnp.dot(M, dinv * hW, preferred_element_type=jnp.float32)
    hfin = hfin + bg_ref[...]
    a = pa_ref[...]
    hfin = jnp.where(hfin >= 0, hfin, a * hfin)
    hfin = jnp.where(jnp.isnan(hfin), 0.0, hfin)
    hfin = jnp.clip(hfin, -_FMAX, _FMAX)
    out_ref[...] = hfin


def kernel(x, zone, region_adjacency, S, Wq, bq, Wk, bk, Wv, bv, Wo, bo,
           Wg, bg, prelu_a):
    f32 = jnp.float32
    x = x.astype(f32)
    x_perm = x.reshape(NQ, NH, H).transpose(1, 0, 2)
    x_perm = jnp.pad(x_perm, ((0, 0), (0, NQP - NQ), (0, 0)))
    zone_q = zone.astype(jnp.int32).reshape(NH, NQ)
    zone_q = jnp.pad(zone_q, ((0, 0), (0, NQP - NQ)), constant_values=-1)
    zone_r = zone_q.reshape(NH, NB, BJ).transpose(1, 0, 2)
    s2 = S.reshape(1, H).astype(f32)
    bq2 = bq.reshape(1, H).astype(f32)
    bk2 = bk.reshape(1, H).astype(f32)
    bv2 = bv.reshape(1, H).astype(f32)
    bo2 = bo.reshape(1, H).astype(f32)
    bg2 = bg.reshape(1, H).astype(f32)
    adj3 = (region_adjacency.astype(jnp.int32)
            .reshape(2, NEC, EC).transpose(1, 0, 2))
    pa = prelu_a.reshape(1, 1).astype(f32)

    num, den = pl.pallas_call(
        _attn_body,
        grid=(NB,),
        in_specs=[
            pl.BlockSpec((NH, BJ, H), lambda i: (0, i, 0)),
            pl.BlockSpec((1, NH, BJ), lambda i: (i, 0, 0)),
            pl.BlockSpec((1, H), lambda i: (0, 0)),
            pl.BlockSpec((H, H), lambda i: (0, 0)),
            pl.BlockSpec((1, H), lambda i: (0, 0)),
            pl.BlockSpec((H, H), lambda i: (0, 0)),
            pl.BlockSpec((1, H), lambda i: (0, 0)),
            pl.BlockSpec((H, H), lambda i: (0, 0)),
            pl.BlockSpec((1, H), lambda i: (0, 0)),
        ],
        out_specs=[
            pl.BlockSpec((R, H), lambda i: (0, 0)),
            pl.BlockSpec((R, NH), lambda i: (0, 0)),
        ],
        out_shape=[
            jax.ShapeDtypeStruct((R, H), f32),
            jax.ShapeDtypeStruct((R, NH), f32),
        ],
    )(x_perm, zone_r, s2, Wq.astype(f32), bq2, Wk.astype(f32), bk2,
      Wv.astype(f32), bv2)

    out = pl.pallas_call(
        _head_body,
        out_shape=jax.ShapeDtypeStruct((R, H), f32),
    )(num, den, s2, Wq.astype(f32), bq2, Wo.astype(f32), bo2,
      Wg.astype(f32), bg2, adj3, pa)
    return out


# speedup vs baseline: 14.9100x; 1.3369x over previous
import jax
import jax.numpy as jnp
from jax.experimental import pallas as pl

N = 50000
H = 256
NH = 4
DS = H // NH
R = 1000
E = 16000
NQ = N // NH
BJ = 500
NB = NQ // BJ
BX = NH * BJ
EC = 1000
NEC = E // EC

_FMAX = 3.4028234663852886e38


def _attn_body(x_ref, zone_ref, s_ref, wq_ref, bq_ref, wk_ref, bk_ref,
               wv_ref, bv_ref, num_ref, den_ref):
    i = pl.program_id(0)

    @pl.when(i == 0)
    def _():
        num_ref[...] = jnp.zeros_like(num_ref)
        den_ref[...] = jnp.zeros_like(den_ref)

    qseed = jnp.dot(s_ref[...], wq_ref[...],
                    preferred_element_type=jnp.float32) + bq_ref[...]

    gi = jax.lax.broadcasted_iota(jnp.int32, (H, NH), 0) // DS
    gj = jax.lax.broadcasted_iota(jnp.int32, (H, NH), 1)
    G = (gi == gj).astype(jnp.float32)

    kfull = jnp.dot(x_ref[...], wk_ref[...],
                    preferred_element_type=jnp.float32) + bk_ref[...]
    vfull = jnp.dot(x_ref[...], wv_ref[...],
                    preferred_element_type=jnp.float32) + bv_ref[...]
    k3 = kfull.reshape(BJ, NH, H)
    v3 = vfull.reshape(BJ, NH, H)

    vws = []
    exs = []
    for h in range(NH):
        kh = k3[:, h, :]
        vh = v3[:, h, :]
        qtile = jnp.tile(qseed[:, h * DS:(h + 1) * DS], (1, NH))
        ph = jnp.dot(kh * qtile, G,
                     preferred_element_type=jnp.float32)
        exh = jnp.exp(ph * (1.0 / 16.0))
        exs.append(exh)
        vws.append([vh[:, g * DS:(g + 1) * DS] * exh[:, g:g + 1]
                    for g in range(NH)])

    iota_r = jax.lax.broadcasted_iota(jnp.int32, (R, BJ), 0)
    zall = zone_ref[0]
    accs = [jnp.zeros((R, DS), jnp.float32) for _ in range(NH)]
    dacc = jnp.zeros((R, NH), jnp.float32)
    for g in range(NH):
        oh = (iota_r == zall[g:g + 1, :]).astype(jnp.float32)
        exg = jnp.concatenate([exs[h][:, g:g + 1] for h in range(NH)],
                              axis=1)
        dacc = dacc + jnp.dot(oh, exg, preferred_element_type=jnp.float32)
        for h in range(NH):
            accs[h] = accs[h] + jnp.dot(oh, vws[h][g],
                                        preferred_element_type=jnp.float32)
    num_ref[...] += jnp.concatenate(accs, axis=1)
    den_ref[...] += dacc


def _head_body(num_ref, den_ref, s_ref, wq_ref, bq_ref, wo_ref, bo_ref,
               wg_ref, bg_ref, adj_ref, pa_ref, out_ref):
    qseed = jnp.dot(s_ref[...], wq_ref[...],
                    preferred_element_type=jnp.float32) + bq_ref[...]

    gi = jax.lax.broadcasted_iota(jnp.int32, (NH, H), 1) // DS
    gj = jax.lax.broadcasted_iota(jnp.int32, (NH, H), 0)
    GT = (gi == gj).astype(jnp.float32)
    denR = jnp.dot(den_ref[...], GT, preferred_element_type=jnp.float32)

    region = num_ref[...] / (denR + 1e-16)
    O = qseed + region
    O = O + jax.nn.relu(jnp.dot(O, wo_ref[...],
                                preferred_element_type=jnp.float32) + bo_ref[...])
    hW = jnp.dot(O, wg_ref[...], preferred_element_type=jnp.float32)

    iota_re = jax.lax.broadcasted_iota(jnp.int32, (R, EC), 0)

    def _edge_chunk(c, M):
        rows = adj_ref[c, 0:1, :]
        cols = adj_ref[c, 1:2, :]
        ohr = (iota_re == rows).astype(jnp.bfloat16)
        ohc = (iota_re == cols).astype(jnp.bfloat16)
        return M + jax.lax.dot_general(
            ohc, ohr, (((1,), (1,)), ((), ())),
            preferred_element_type=jnp.float32)

    M = jax.lax.fori_loop(0, NEC, _edge_chunk,
                          jnp.zeros((R, R), jnp.float32))
    ri = jax.lax.broadcasted_iota(jnp.int32, (R, R), 0)
    rj = jax.lax.broadcasted_iota(jnp.int32, (R, R), 1)
    M = M + (ri == rj).astype(jnp.float32)

    deg = jnp.sum(M, axis=1, keepdims=True)
    dinv = jax.lax.rsqrt(deg)
    hfin = dinv * jnp.dot(M, dinv * hW, preferred_element_type=jnp.float32)
    hfin = hfin + bg_ref[...]
    a = pa_ref[...]
    hfin = jnp.where(hfin >= 0, hfin, a * hfin)
    hfin = jnp.where(jnp.isnan(hfin), 0.0, hfin)
    hfin = jnp.clip(hfin, -_FMAX, _FMAX)
    out_ref[...] = hfin


def kernel(x, zone, region_adjacency, S, Wq, bq, Wk, bk, Wv, bv, Wo, bo,
           Wg, bg, prelu_a):
    f32 = jnp.float32
    x = x.astype(f32)
    zone_r = zone.astype(jnp.int32).reshape(NH, NB, BJ).transpose(1, 0, 2)
    s2 = S.reshape(1, H).astype(f32)
    bq2 = bq.reshape(1, H).astype(f32)
    bk2 = bk.reshape(1, H).astype(f32)
    bv2 = bv.reshape(1, H).astype(f32)
    bo2 = bo.reshape(1, H).astype(f32)
    bg2 = bg.reshape(1, H).astype(f32)
    adj3 = (region_adjacency.astype(jnp.int32)
            .reshape(2, NEC, EC).transpose(1, 0, 2))
    pa = prelu_a.reshape(1, 1).astype(f32)

    num, den = pl.pallas_call(
        _attn_body,
        grid=(NB,),
        in_specs=[
            pl.BlockSpec((BX, H), lambda i: (i, 0)),
            pl.BlockSpec((1, NH, BJ), lambda i: (i, 0, 0)),
            pl.BlockSpec((1, H), lambda i: (0, 0)),
            pl.BlockSpec((H, H), lambda i: (0, 0)),
            pl.BlockSpec((1, H), lambda i: (0, 0)),
            pl.BlockSpec((H, H), lambda i: (0, 0)),
            pl.BlockSpec((1, H), lambda i: (0, 0)),
            pl.BlockSpec((H, H), lambda i: (0, 0)),
            pl.BlockSpec((1, H), lambda i: (0, 0)),
        ],
        out_specs=[
            pl.BlockSpec((R, H), lambda i: (0, 0)),
            pl.BlockSpec((R, NH), lambda i: (0, 0)),
        ],
        out_shape=[
            jax.ShapeDtypeStruct((R, H), f32),
            jax.ShapeDtypeStruct((R, NH), f32),
        ],
    )(x, zone_r, s2, Wq.astype(f32), bq2, Wk.astype(f32), bk2,
      Wv.astype(f32), bv2)

    out = pl.pallas_call(
        _head_body,
        out_shape=jax.ShapeDtypeStruct((R, H), f32),
    )(num, den, s2, Wq.astype(f32), bq2, Wo.astype(f32), bo2,
      Wg.astype(f32), bg2, adj3, pa)
    return out


# speedup vs baseline: 17.1009x; 1.1469x over previous
import jax
import jax.numpy as jnp
from jax.experimental import pallas as pl

N = 50000
H = 256
NH = 4
DS = H // NH
R = 1000
E = 16000
NQ = N // NH
BJ = 500
NB = NQ // BJ
BX = NH * BJ
EC = 1000
NEC = E // EC

_FMAX = 3.4028234663852886e38


def _attn_body(x_ref, zz_ref, s_ref, wq_ref, bq_ref, wk_ref, bk_ref,
               wv_ref, bv_ref, num_ref, den_ref):
    i = pl.program_id(0)

    @pl.when(i == 0)
    def _():
        num_ref[...] = jnp.zeros_like(num_ref)
        den_ref[...] = jnp.zeros_like(den_ref)

    qseed = jnp.dot(s_ref[...], wq_ref[...],
                    preferred_element_type=jnp.float32) + bq_ref[...]

    gi = jax.lax.broadcasted_iota(jnp.int32, (H, NH), 0) // DS
    gj = jax.lax.broadcasted_iota(jnp.int32, (H, NH), 1)
    G = (gi == gj).astype(jnp.float32)
    ti = jax.lax.broadcasted_iota(jnp.int32, (NH, H), 1) // DS
    tj = jax.lax.broadcasted_iota(jnp.int32, (NH, H), 0)
    GT = (ti == tj).astype(jnp.float32)

    kfull = jnp.dot(x_ref[...], wk_ref[...],
                    preferred_element_type=jnp.float32) + bk_ref[...]
    vfull = jnp.dot(x_ref[...], wv_ref[...],
                    preferred_element_type=jnp.float32) + bv_ref[...]

    rowmod = jax.lax.broadcasted_iota(jnp.int32, (BX, H), 0) % NH
    QB = jnp.zeros((BX, H), jnp.float32)
    for h in range(NH):
        qtile = jnp.tile(qseed[:, h * DS:(h + 1) * DS], (1, NH))
        QB = jnp.where(rowmod == h, qtile, QB)

    P = jnp.dot(kfull * QB, G, preferred_element_type=jnp.float32)
    exall = jnp.exp(P * (1.0 / 16.0))
    VWr = vfull * jnp.dot(exall, GT, preferred_element_type=jnp.float32)

    vw4 = VWr.reshape(BJ, NH, H)
    ex4 = exall.reshape(BJ, NH, NH)
    VRs = [jnp.concatenate([vw4[:, h, g * DS:(g + 1) * DS]
                            for g in range(NH)], axis=0)
           for h in range(NH)]
    EXCAT = jnp.concatenate(
        [jnp.concatenate([ex4[:, h, g:g + 1] for g in range(NH)], axis=0)
         for h in range(NH)], axis=1)

    iota_r = jax.lax.broadcasted_iota(jnp.int32, (R, BX), 0)
    OH = (iota_r == zz_ref[0]).astype(jnp.float32)
    den_ref[...] += jnp.dot(OH, EXCAT, preferred_element_type=jnp.float32)
    num_ref[...] += jnp.concatenate(
        [jnp.dot(OH, VRs[h], preferred_element_type=jnp.float32)
         for h in range(NH)], axis=1)


def _head_body(num_ref, den_ref, s_ref, wq_ref, bq_ref, wo_ref, bo_ref,
               wg_ref, bg_ref, adj_ref, pa_ref, out_ref):
    qseed = jnp.dot(s_ref[...], wq_ref[...],
                    preferred_element_type=jnp.float32) + bq_ref[...]

    gi = jax.lax.broadcasted_iota(jnp.int32, (NH, H), 1) // DS
    gj = jax.lax.broadcasted_iota(jnp.int32, (NH, H), 0)
    GT = (gi == gj).astype(jnp.float32)
    denR = jnp.dot(den_ref[...], GT, preferred_element_type=jnp.float32)

    region = num_ref[...] / (denR + 1e-16)
    O = qseed + region
    O = O + jax.nn.relu(jnp.dot(O, wo_ref[...],
                                preferred_element_type=jnp.float32) + bo_ref[...])
    hW = jnp.dot(O, wg_ref[...], preferred_element_type=jnp.float32)

    iota_re = jax.lax.broadcasted_iota(jnp.int32, (R, EC), 0)

    def _edge_chunk(c, M):
        rows = adj_ref[c, 0:1, :]
        cols = adj_ref[c, 1:2, :]
        ohr = (iota_re == rows).astype(jnp.bfloat16)
        ohc = (iota_re == cols).astype(jnp.bfloat16)
        return M + jax.lax.dot_general(
            ohc, ohr, (((1,), (1,)), ((), ())),
            preferred_element_type=jnp.float32)

    M = jax.lax.fori_loop(0, NEC, _edge_chunk,
                          jnp.zeros((R, R), jnp.float32))
    ri = jax.lax.broadcasted_iota(jnp.int32, (R, R), 0)
    rj = jax.lax.broadcasted_iota(jnp.int32, (R, R), 1)
    M = M + (ri == rj).astype(jnp.float32)

    deg = jnp.sum(M, axis=1, keepdims=True)
    dinv = jax.lax.rsqrt(deg)
    hfin = dinv * jnp.dot(M, dinv * hW, preferred_element_type=jnp.float32)
    hfin = hfin + bg_ref[...]
    a = pa_ref[...]
    hfin = jnp.where(hfin >= 0, hfin, a * hfin)
    hfin = jnp.where(jnp.isnan(hfin), 0.0, hfin)
    hfin = jnp.clip(hfin, -_FMAX, _FMAX)
    out_ref[...] = hfin


def kernel(x, zone, region_adjacency, S, Wq, bq, Wk, bk, Wv, bv, Wo, bo,
           Wg, bg, prelu_a):
    f32 = jnp.float32
    x = x.astype(f32)
    zz = (zone.astype(jnp.int32).reshape(NH, NB, BJ)
          .transpose(1, 0, 2).reshape(NB, 1, BX))
    s2 = S.reshape(1, H).astype(f32)
    bq2 = bq.reshape(1, H).astype(f32)
    bk2 = bk.reshape(1, H).astype(f32)
    bv2 = bv.reshape(1, H).astype(f32)
    bo2 = bo.reshape(1, H).astype(f32)
    bg2 = bg.reshape(1, H).astype(f32)
    adj3 = (region_adjacency.astype(jnp.int32)
            .reshape(2, NEC, EC).transpose(1, 0, 2))
    pa = prelu_a.reshape(1, 1).astype(f32)

    num, den = pl.pallas_call(
        _attn_body,
        grid=(NB,),
        in_specs=[
            pl.BlockSpec((BX, H), lambda i: (i, 0)),
            pl.BlockSpec((1, 1, BX), lambda i: (i, 0, 0)),
            pl.BlockSpec((1, H), lambda i: (0, 0)),
            pl.BlockSpec((H, H), lambda i: (0, 0)),
            pl.BlockSpec((1, H), lambda i: (0, 0)),
            pl.BlockSpec((H, H), lambda i: (0, 0)),
            pl.BlockSpec((1, H), lambda i: (0, 0)),
            pl.BlockSpec((H, H), lambda i: (0, 0)),
            pl.BlockSpec((1, H), lambda i: (0, 0)),
        ],
        out_specs=[
            pl.BlockSpec((R, H), lambda i: (0, 0)),
            pl.BlockSpec((R, NH), lambda i: (0, 0)),
        ],
        out_shape=[
            jax.ShapeDtypeStruct((R, H), f32),
            jax.ShapeDtypeStruct((R, NH), f32),
        ],
    )(x, zz, s2, Wq.astype(f32), bq2, Wk.astype(f32), bk2,
      Wv.astype(f32), bv2)

    out = pl.pallas_call(
        _head_body,
        out_shape=jax.ShapeDtypeStruct((R, H), f32),
    )(num, den, s2, Wq.astype(f32), bq2, Wo.astype(f32), bo2,
      Wg.astype(f32), bg2, adj3, pa)
    return out


# speedup vs baseline: 17.1450x; 1.0026x over previous
import functools

import jax
import jax.numpy as jnp
from jax import lax
from jax.experimental import pallas as pl
from jax.experimental.pallas import tpu as pltpu
from jax.experimental.pallas import tpu_sc as plsc

N = 50000
H = 256
NH = 4
DS = H // NH
R = 1000
E = 16000
NQ = N // NH
BJ = 500
NB = NQ // BJ
BX = NH * BJ
EC = 1000
NEC = E // EC

_FMAX = 3.4028234663852886e38


def _attn_body(x_ref, zz_ref, s_ref, wq_ref, bq_ref, wk_ref, bk_ref,
               wv_ref, bv_ref, num_ref, den_ref):
    i = pl.program_id(0)

    @pl.when(i == 0)
    def _():
        num_ref[...] = jnp.zeros_like(num_ref)
        den_ref[...] = jnp.zeros_like(den_ref)

    qseed = jnp.dot(s_ref[...], wq_ref[...],
                    preferred_element_type=jnp.float32) + bq_ref[...]

    gi = jax.lax.broadcasted_iota(jnp.int32, (H, NH), 0) // DS
    gj = jax.lax.broadcasted_iota(jnp.int32, (H, NH), 1)
    G = (gi == gj).astype(jnp.float32)
    ti = jax.lax.broadcasted_iota(jnp.int32, (NH, H), 1) // DS
    tj = jax.lax.broadcasted_iota(jnp.int32, (NH, H), 0)
    GT = (ti == tj).astype(jnp.float32)

    kfull = jnp.dot(x_ref[...], wk_ref[...],
                    preferred_element_type=jnp.float32) + bk_ref[...]
    vfull = jnp.dot(x_ref[...], wv_ref[...],
                    preferred_element_type=jnp.float32) + bv_ref[...]

    rowmod = jax.lax.broadcasted_iota(jnp.int32, (BX, H), 0) % NH
    QB = jnp.zeros((BX, H), jnp.float32)
    for h in range(NH):
        qtile = jnp.tile(qseed[:, h * DS:(h + 1) * DS], (1, NH))
        QB = jnp.where(rowmod == h, qtile, QB)

    P = jnp.dot(kfull * QB, G, preferred_element_type=jnp.float32)
    exall = jnp.exp(P * (1.0 / 16.0))
    VWr = vfull * jnp.dot(exall, GT, preferred_element_type=jnp.float32)

    vw4 = VWr.reshape(BJ, NH, H)
    ex4 = exall.reshape(BJ, NH, NH)
    VRs = [jnp.concatenate([vw4[:, h, g * DS:(g + 1) * DS]
                            for g in range(NH)], axis=0)
           for h in range(NH)]
    EXCAT = jnp.concatenate(
        [jnp.concatenate([ex4[:, h, g:g + 1] for g in range(NH)], axis=0)
         for h in range(NH)], axis=1)

    iota_r = jax.lax.broadcasted_iota(jnp.int32, (R, BX), 0)
    OH = (iota_r == zz_ref[0]).astype(jnp.float32)
    den_ref[...] += jnp.dot(OH, EXCAT, preferred_element_type=jnp.float32)
    num_ref[...] += jnp.concatenate(
        [jnp.dot(OH, VRs[h], preferred_element_type=jnp.float32)
         for h in range(NH)], axis=1)


WACC = 128
RACC = 4096
CH = 400
CPA = N // CH
NCHUNK = NH * CPA
SUB = 100


def _attn_sc_body(x_ref, zz_ref, s_ref, wq_ref, bq_ref, wk_ref, bk_ref,
                  wv_ref, bv_ref, vrx_ref):
    qseed = jnp.dot(s_ref[...], wq_ref[...],
                    preferred_element_type=jnp.float32) + bq_ref[...]
    gi = jax.lax.broadcasted_iota(jnp.int32, (H, NH), 0) // DS
    gj = jax.lax.broadcasted_iota(jnp.int32, (H, NH), 1)
    G = (gi == gj).astype(jnp.float32)
    ti = jax.lax.broadcasted_iota(jnp.int32, (NH, H), 1) // DS
    tj = jax.lax.broadcasted_iota(jnp.int32, (NH, H), 0)
    GT = (ti == tj).astype(jnp.float32)

    kfull = jnp.dot(x_ref[...], wk_ref[...],
                    preferred_element_type=jnp.float32) + bk_ref[...]
    vfull = jnp.dot(x_ref[...], wv_ref[...],
                    preferred_element_type=jnp.float32) + bv_ref[...]

    rowmod = jax.lax.broadcasted_iota(jnp.int32, (BX, H), 0) % NH
    QB = jnp.zeros((BX, H), jnp.float32)
    for h in range(NH):
        qtile = jnp.tile(qseed[:, h * DS:(h + 1) * DS], (1, NH))
        QB = jnp.where(rowmod == h, qtile, QB)

    P = jnp.dot(kfull * QB, G, preferred_element_type=jnp.float32)
    exall = jnp.exp(P * (1.0 / 16.0))
    VWr = vfull * jnp.dot(exall, GT, preferred_element_type=jnp.float32)

    vw4 = VWr.reshape(BJ, NH, H)
    ex4 = exall.reshape(BJ, NH, NH)
    pad = jnp.zeros((BX, WACC - DS - 1), jnp.float32)
    for h in range(NH):
        vr = jnp.concatenate([vw4[:, h, g * DS:(g + 1) * DS]
                              for g in range(NH)], axis=0)
        exc = jnp.concatenate([ex4[:, h, g:g + 1] for g in range(NH)],
                              axis=0)
        vrx_ref[h] = jnp.concatenate([vr, exc, pad], axis=1)


def _sc_segsum(vr_hbm, tgt_hbm, zeros_hbm, out_hbm, idx_v, rows_v, acc_sh):
    cid = lax.axis_index("c")
    sid = lax.axis_index("s")
    w = sid * 2 + cid

    @pl.when(sid == 0)
    def _():
        pltpu.sync_copy(zeros_hbm, acc_sh)
    plsc.subcore_barrier()

    for k in range(16):
        q = w + 32 * k

        @pl.when(q < NCHUNK)
        def _():
            g = q // CPA
            c = q % CPA
            pltpu.sync_copy(tgt_hbm.at[g, c], idx_v)
            pltpu.sync_copy(vr_hbm.at[g, pl.ds(c * CH, CH)], rows_v)
            for b in range(CH // SUB):
                pltpu.sync_copy(rows_v.at[pl.ds(b * SUB, SUB)],
                                acc_sh.at[idx_v.at[b]], add=True)

    plsc.subcore_barrier()

    @pl.when(sid == 0)
    def _():
        pltpu.sync_copy(acc_sh, out_hbm.at[cid])


def _head_body(acc2_ref, s_ref, wq_ref, bq_ref, wo_ref, bo_ref,
               wg_ref, bg_ref, adj_ref, pa_ref, out_ref):
    qseed = jnp.dot(s_ref[...], wq_ref[...],
                    preferred_element_type=jnp.float32) + bq_ref[...]

    a = acc2_ref[0] + acc2_ref[1]
    num = jnp.concatenate([a[h * R:(h + 1) * R, 0:DS] for h in range(NH)],
                          axis=1)
    den4 = jnp.concatenate([a[h * R:(h + 1) * R, DS:DS + 1]
                            for h in range(NH)], axis=1)

    gi = jax.lax.broadcasted_iota(jnp.int32, (NH, H), 1) // DS
    gj = jax.lax.broadcasted_iota(jnp.int32, (NH, H), 0)
    GT = (gi == gj).astype(jnp.float32)
    denR = jnp.dot(den4, GT, preferred_element_type=jnp.float32)

    region = num / (denR + 1e-16)
    O = qseed + region
    O = O + jax.nn.relu(jnp.dot(O, wo_ref[...],
                                preferred_element_type=jnp.float32) + bo_ref[...])
    hW = jnp.dot(O, wg_ref[...], preferred_element_type=jnp.float32)

    iota_re = jax.lax.broadcasted_iota(jnp.int32, (R, EC), 0)

    def _edge_chunk(c, M):
        rows = adj_ref[c, 0:1, :]
        cols = adj_ref[c, 1:2, :]
        ohr = (iota_re == rows).astype(jnp.bfloat16)
        ohc = (iota_re == cols).astype(jnp.bfloat16)
        return M + jax.lax.dot_general(
            ohc, ohr, (((1,), (1,)), ((), ())),
            preferred_element_type=jnp.float32)

    M = jax.lax.fori_loop(0, NEC, _edge_chunk,
                          jnp.zeros((R, R), jnp.float32))
    ri = jax.lax.broadcasted_iota(jnp.int32, (R, R), 0)
    rj = jax.lax.broadcasted_iota(jnp.int32, (R, R), 1)
    M = M + (ri == rj).astype(jnp.float32)

    deg = jnp.sum(M, axis=1, keepdims=True)
    dinv = jax.lax.rsqrt(deg)
    hfin = dinv * jnp.dot(M, dinv * hW, preferred_element_type=jnp.float32)
    hfin = hfin + bg_ref[...]
    a = pa_ref[...]
    hfin = jnp.where(hfin >= 0, hfin, a * hfin)
    hfin = jnp.where(jnp.isnan(hfin), 0.0, hfin)
    hfin = jnp.clip(hfin, -_FMAX, _FMAX)
    out_ref[...] = hfin


def kernel(x, zone, region_adjacency, S, Wq, bq, Wk, bk, Wv, bv, Wo, bo,
           Wg, bg, prelu_a):
    f32 = jnp.float32
    x = x.astype(f32)
    zz = (zone.astype(jnp.int32).reshape(NH, NB, BJ)
          .transpose(1, 0, 2).reshape(NB, 1, BX))
    s2 = S.reshape(1, H).astype(f32)
    bq2 = bq.reshape(1, H).astype(f32)
    bk2 = bk.reshape(1, H).astype(f32)
    bv2 = bv.reshape(1, H).astype(f32)
    bo2 = bo.reshape(1, H).astype(f32)
    bg2 = bg.reshape(1, H).astype(f32)
    adj3 = (region_adjacency.astype(jnp.int32)
            .reshape(2, NEC, EC).transpose(1, 0, 2))
    pa = prelu_a.reshape(1, 1).astype(f32)

    vrx = pl.pallas_call(
        _attn_sc_body,
        grid=(NB,),
        in_specs=[
            pl.BlockSpec((BX, H), lambda i: (i, 0)),
            pl.BlockSpec((1, 1, BX), lambda i: (i, 0, 0)),
            pl.BlockSpec((1, H), lambda i: (0, 0)),
            pl.BlockSpec((H, H), lambda i: (0, 0)),
            pl.BlockSpec((1, H), lambda i: (0, 0)),
            pl.BlockSpec((H, H), lambda i: (0, 0)),
            pl.BlockSpec((1, H), lambda i: (0, 0)),
            pl.BlockSpec((H, H), lambda i: (0, 0)),
            pl.BlockSpec((1, H), lambda i: (0, 0)),
        ],
        out_specs=pl.BlockSpec((NH, BX, WACC), lambda i: (0, i, 0)),
        out_shape=jax.ShapeDtypeStruct((NH, N, WACC), f32),
    )(x, zz, s2, Wq.astype(f32), bq2, Wk.astype(f32), bk2,
      Wv.astype(f32), bv2)

    tgt = (zz.reshape(1, N)
           + R * jnp.arange(NH, dtype=jnp.int32)[:, None]
           ).reshape(NH, CPA, CH // SUB, SUB)
    zeros2 = jnp.zeros((RACC, WACC), f32)
    mesh = plsc.VectorSubcoreMesh(core_axis_name="c", subcore_axis_name="s")
    seg = pl.kernel(
        _sc_segsum,
        mesh=mesh,
        out_type=jax.ShapeDtypeStruct((2, RACC, WACC), f32),
        scratch_types=[
            pltpu.VMEM((CH // SUB, SUB), jnp.int32),
            pltpu.VMEM((CH, WACC), f32),
            pltpu.VMEM_SHARED((RACC, WACC), f32),
        ],
    )
    acc2 = seg(vrx, tgt, zeros2)

    out = pl.pallas_call(
        _head_body,
        out_shape=jax.ShapeDtypeStruct((R, H), f32),
    )(acc2, s2, Wq.astype(f32), bq2, Wo.astype(f32), bo2,
      Wg.astype(f32), bg2, adj3, pa)
    return out


# speedup vs baseline: 20.0079x; 1.1670x over previous
import functools

import jax
import jax.numpy as jnp
from jax import lax
from jax.experimental import pallas as pl
from jax.experimental.pallas import tpu as pltpu
from jax.experimental.pallas import tpu_sc as plsc

N = 50000
H = 256
NH = 4
DS = H // NH
R = 1000
E = 16000
NQ = N // NH
BJ = 500
NB = NQ // BJ
BX = NH * BJ
EC = 1000
NEC = E // EC

_FMAX = 3.4028234663852886e38


def _attn_body(x_ref, zz_ref, s_ref, wq_ref, bq_ref, wk_ref, bk_ref,
               wv_ref, bv_ref, num_ref, den_ref):
    i = pl.program_id(0)

    @pl.when(i == 0)
    def _():
        num_ref[...] = jnp.zeros_like(num_ref)
        den_ref[...] = jnp.zeros_like(den_ref)

    qseed = jnp.dot(s_ref[...], wq_ref[...],
                    preferred_element_type=jnp.float32) + bq_ref[...]

    gi = jax.lax.broadcasted_iota(jnp.int32, (H, NH), 0) // DS
    gj = jax.lax.broadcasted_iota(jnp.int32, (H, NH), 1)
    G = (gi == gj).astype(jnp.float32)
    ti = jax.lax.broadcasted_iota(jnp.int32, (NH, H), 1) // DS
    tj = jax.lax.broadcasted_iota(jnp.int32, (NH, H), 0)
    GT = (ti == tj).astype(jnp.float32)

    kfull = jnp.dot(x_ref[...], wk_ref[...],
                    preferred_element_type=jnp.float32) + bk_ref[...]
    vfull = jnp.dot(x_ref[...], wv_ref[...],
                    preferred_element_type=jnp.float32) + bv_ref[...]

    rowmod = jax.lax.broadcasted_iota(jnp.int32, (BX, H), 0) % NH
    QB = jnp.zeros((BX, H), jnp.float32)
    for h in range(NH):
        qtile = jnp.tile(qseed[:, h * DS:(h + 1) * DS], (1, NH))
        QB = jnp.where(rowmod == h, qtile, QB)

    P = jnp.dot(kfull * QB, G, preferred_element_type=jnp.float32)
    exall = jnp.exp(P * (1.0 / 16.0))
    VWr = vfull * jnp.dot(exall, GT, preferred_element_type=jnp.float32)

    vw4 = VWr.reshape(BJ, NH, H)
    ex4 = exall.reshape(BJ, NH, NH)
    VRs = [jnp.concatenate([vw4[:, h, g * DS:(g + 1) * DS]
                            for g in range(NH)], axis=0)
           for h in range(NH)]
    EXCAT = jnp.concatenate(
        [jnp.concatenate([ex4[:, h, g:g + 1] for g in range(NH)], axis=0)
         for h in range(NH)], axis=1)

    iota_r = jax.lax.broadcasted_iota(jnp.int32, (R, BX), 0)
    OH = (iota_r == zz_ref[0]).astype(jnp.float32)
    den_ref[...] += jnp.dot(OH, EXCAT, preferred_element_type=jnp.float32)
    num_ref[...] += jnp.concatenate(
        [jnp.dot(OH, VRs[h], preferred_element_type=jnp.float32)
         for h in range(NH)], axis=1)


WACC = 128
RACC = 4096
CH = 400
CPA = N // CH
NCHUNK = NH * CPA
SUB = 100


def _attn_sc_body(x_ref, zz_ref, s_ref, wq_ref, bq_ref, wk_ref, bk_ref,
                  wv_ref, bv_ref, vrx_ref):
    qseed = jnp.dot(s_ref[...], wq_ref[...],
                    preferred_element_type=jnp.float32) + bq_ref[...]
    gi = jax.lax.broadcasted_iota(jnp.int32, (H, NH), 0) // DS
    gj = jax.lax.broadcasted_iota(jnp.int32, (H, NH), 1)
    G = (gi == gj).astype(jnp.float32)
    ti = jax.lax.broadcasted_iota(jnp.int32, (NH, H), 1) // DS
    tj = jax.lax.broadcasted_iota(jnp.int32, (NH, H), 0)
    GT = (ti == tj).astype(jnp.float32)

    kfull = jnp.dot(x_ref[...], wk_ref[...],
                    preferred_element_type=jnp.float32) + bk_ref[...]
    vfull = jnp.dot(x_ref[...], wv_ref[...],
                    preferred_element_type=jnp.float32) + bv_ref[...]

    rowmod = jax.lax.broadcasted_iota(jnp.int32, (BX, H), 0) % NH
    QB = jnp.zeros((BX, H), jnp.float32)
    for h in range(NH):
        qtile = jnp.tile(qseed[:, h * DS:(h + 1) * DS], (1, NH))
        QB = jnp.where(rowmod == h, qtile, QB)

    P = jnp.dot(kfull * QB, G, preferred_element_type=jnp.float32)
    exall = jnp.exp(P * (1.0 / 16.0))
    VWr = vfull * jnp.dot(exall, GT, preferred_element_type=jnp.float32)

    vw4 = VWr.reshape(BJ, NH, H)
    ex4 = exall.reshape(BJ, NH, NH)
    pad = jnp.zeros((BX, WACC - DS - 1), jnp.float32)
    for h in range(NH):
        vr = jnp.concatenate([vw4[:, h, g * DS:(g + 1) * DS]
                              for g in range(NH)], axis=0)
        exc = jnp.concatenate([ex4[:, h, g:g + 1] for g in range(NH)],
                              axis=0)
        vrx_ref[h] = jnp.concatenate([vr, exc, pad], axis=1)


def _sc_segsum(vr_hbm, tgt_hbm, zeros_hbm, out_hbm, idx_v, rows_v, acc_sh):
    cid = lax.axis_index("c")
    sid = lax.axis_index("s")
    w = sid * 2 + cid

    @pl.when(sid == 0)
    def _():
        pltpu.sync_copy(zeros_hbm, acc_sh)
    plsc.subcore_barrier()

    for k in range(16):
        q = w + 32 * k

        @pl.when(q < NCHUNK)
        def _():
            g = q // CPA
            c = q % CPA
            pltpu.sync_copy(tgt_hbm.at[g, c], idx_v)
            pltpu.sync_copy(vr_hbm.at[g, pl.ds(c * CH, CH)], rows_v)
            for b in range(CH // SUB):
                pltpu.sync_copy(rows_v.at[pl.ds(b * SUB, SUB)],
                                acc_sh.at[idx_v.at[b]], add=True)

    plsc.subcore_barrier()

    @pl.when(sid == 0)
    def _():
        pltpu.sync_copy(acc_sh, out_hbm.at[cid])


def _gcn_mat_body(adj_ref, m_ref):
    iota_re = jax.lax.broadcasted_iota(jnp.int32, (R, EC), 0)

    def _edge_chunk(c, M):
        rows = adj_ref[c, 0:1, :]
        cols = adj_ref[c, 1:2, :]
        ohr = (iota_re == rows).astype(jnp.bfloat16)
        ohc = (iota_re == cols).astype(jnp.bfloat16)
        return M + jax.lax.dot_general(
            ohc, ohr, (((1,), (1,)), ((), ())),
            preferred_element_type=jnp.float32)

    M = jax.lax.fori_loop(0, NEC, _edge_chunk,
                          jnp.zeros((R, R), jnp.float32))
    ri = jax.lax.broadcasted_iota(jnp.int32, (R, R), 0)
    rj = jax.lax.broadcasted_iota(jnp.int32, (R, R), 1)
    m_ref[...] = M + (ri == rj).astype(jnp.float32)


def _head_body(acc2_ref, m_ref, s_ref, wq_ref, bq_ref, wo_ref, bo_ref,
               wg_ref, bg_ref, pa_ref, out_ref):
    qseed = jnp.dot(s_ref[...], wq_ref[...],
                    preferred_element_type=jnp.float32) + bq_ref[...]

    a = acc2_ref[0] + acc2_ref[1]
    num = jnp.concatenate([a[h * R:(h + 1) * R, 0:DS] for h in range(NH)],
                          axis=1)
    den4 = jnp.concatenate([a[h * R:(h + 1) * R, DS:DS + 1]
                            for h in range(NH)], axis=1)

    gi = jax.lax.broadcasted_iota(jnp.int32, (NH, H), 1) // DS
    gj = jax.lax.broadcasted_iota(jnp.int32, (NH, H), 0)
    GT = (gi == gj).astype(jnp.float32)
    denR = jnp.dot(den4, GT, preferred_element_type=jnp.float32)

    region = num / (denR + 1e-16)
    O = qseed + region
    O = O + jax.nn.relu(jnp.dot(O, wo_ref[...],
                                preferred_element_type=jnp.float32) + bo_ref[...])
    hW = jnp.dot(O, wg_ref[...], preferred_element_type=jnp.float32)

    M = m_ref[...]
    deg = jnp.sum(M, axis=1, keepdims=True)
    dinv = jax.lax.rsqrt(deg)
    hfin = dinv * jnp.dot(M, dinv * hW, preferred_element_type=jnp.float32)
    hfin = hfin + bg_ref[...]
    a = pa_ref[...]
    hfin = jnp.where(hfin >= 0, hfin, a * hfin)
    hfin = jnp.where(jnp.isnan(hfin), 0.0, hfin)
    hfin = jnp.clip(hfin, -_FMAX, _FMAX)
    out_ref[...] = hfin


def kernel(x, zone, region_adjacency, S, Wq, bq, Wk, bk, Wv, bv, Wo, bo,
           Wg, bg, prelu_a):
    f32 = jnp.float32
    x = x.astype(f32)
    zz = (zone.astype(jnp.int32).reshape(NH, NB, BJ)
          .transpose(1, 0, 2).reshape(NB, 1, BX))
    s2 = S.reshape(1, H).astype(f32)
    bq2 = bq.reshape(1, H).astype(f32)
    bk2 = bk.reshape(1, H).astype(f32)
    bv2 = bv.reshape(1, H).astype(f32)
    bo2 = bo.reshape(1, H).astype(f32)
    bg2 = bg.reshape(1, H).astype(f32)
    adj3 = (region_adjacency.astype(jnp.int32)
            .reshape(2, NEC, EC).transpose(1, 0, 2))
    pa = prelu_a.reshape(1, 1).astype(f32)

    vrx = pl.pallas_call(
        _attn_sc_body,
        grid=(NB,),
        in_specs=[
            pl.BlockSpec((BX, H), lambda i: (i, 0)),
            pl.BlockSpec((1, 1, BX), lambda i: (i, 0, 0)),
            pl.BlockSpec((1, H), lambda i: (0, 0)),
            pl.BlockSpec((H, H), lambda i: (0, 0)),
            pl.BlockSpec((1, H), lambda i: (0, 0)),
            pl.BlockSpec((H, H), lambda i: (0, 0)),
            pl.BlockSpec((1, H), lambda i: (0, 0)),
            pl.BlockSpec((H, H), lambda i: (0, 0)),
            pl.BlockSpec((1, H), lambda i: (0, 0)),
        ],
        out_specs=pl.BlockSpec((NH, BX, WACC), lambda i: (0, i, 0)),
        out_shape=jax.ShapeDtypeStruct((NH, N, WACC), f32),
    )(x, zz, s2, Wq.astype(f32), bq2, Wk.astype(f32), bk2,
      Wv.astype(f32), bv2)

    tgt = (zz.reshape(1, N)
           + R * jnp.arange(NH, dtype=jnp.int32)[:, None]
           ).reshape(NH, CPA, CH // SUB, SUB)
    zeros2 = jnp.zeros((RACC, WACC), f32)
    mesh = plsc.VectorSubcoreMesh(core_axis_name="c", subcore_axis_name="s")
    seg = pl.kernel(
        _sc_segsum,
        mesh=mesh,
        out_type=jax.ShapeDtypeStruct((2, RACC, WACC), f32),
        scratch_types=[
            pltpu.VMEM((CH // SUB, SUB), jnp.int32),
            pltpu.VMEM((CH, WACC), f32),
            pltpu.VMEM_SHARED((RACC, WACC), f32),
        ],
    )
    m_mat = pl.pallas_call(
        _gcn_mat_body,
        out_shape=jax.ShapeDtypeStruct((R, R), f32),
    )(adj3)

    acc2 = seg(vrx, tgt, zeros2)

    out = pl.pallas_call(
        _head_body,
        out_shape=jax.ShapeDtypeStruct((R, H), f32),
    )(acc2, m_mat, s2, Wq.astype(f32), bq2, Wo.astype(f32), bo2,
      Wg.astype(f32), bg2, pa)
    return out


# speedup vs baseline: 22.3371x; 1.1164x over previous
import functools

import jax
import jax.numpy as jnp
from jax import lax
from jax.experimental import pallas as pl
from jax.experimental.pallas import tpu as pltpu
from jax.experimental.pallas import tpu_sc as plsc

N = 50000
H = 256
NH = 4
DS = H // NH
R = 1000
E = 16000
NQ = N // NH
BJ = 500
NB = NQ // BJ
BX = NH * BJ
EC = 1000
NEC = E // EC

_FMAX = 3.4028234663852886e38


def _attn_body(x_ref, zz_ref, s_ref, wq_ref, bq_ref, wk_ref, bk_ref,
               wv_ref, bv_ref, num_ref, den_ref):
    i = pl.program_id(0)

    @pl.when(i == 0)
    def _():
        num_ref[...] = jnp.zeros_like(num_ref)
        den_ref[...] = jnp.zeros_like(den_ref)

    qseed = jnp.dot(s_ref[...], wq_ref[...],
                    preferred_element_type=jnp.float32) + bq_ref[...]

    gi = jax.lax.broadcasted_iota(jnp.int32, (H, NH), 0) // DS
    gj = jax.lax.broadcasted_iota(jnp.int32, (H, NH), 1)
    G = (gi == gj).astype(jnp.float32)
    ti = jax.lax.broadcasted_iota(jnp.int32, (NH, H), 1) // DS
    tj = jax.lax.broadcasted_iota(jnp.int32, (NH, H), 0)
    GT = (ti == tj).astype(jnp.float32)

    kfull = jnp.dot(x_ref[...], wk_ref[...],
                    preferred_element_type=jnp.float32) + bk_ref[...]
    vfull = jnp.dot(x_ref[...], wv_ref[...],
                    preferred_element_type=jnp.float32) + bv_ref[...]

    rowmod = jax.lax.broadcasted_iota(jnp.int32, (BX, H), 0) % NH
    QB = jnp.zeros((BX, H), jnp.float32)
    for h in range(NH):
        qtile = jnp.tile(qseed[:, h * DS:(h + 1) * DS], (1, NH))
        QB = jnp.where(rowmod == h, qtile, QB)

    P = jnp.dot(kfull * QB, G, preferred_element_type=jnp.float32)
    exall = jnp.exp(P * (1.0 / 16.0))
    VWr = vfull * jnp.dot(exall, GT, preferred_element_type=jnp.float32)

    vw4 = VWr.reshape(BJ, NH, H)
    ex4 = exall.reshape(BJ, NH, NH)
    VRs = [jnp.concatenate([vw4[:, h, g * DS:(g + 1) * DS]
                            for g in range(NH)], axis=0)
           for h in range(NH)]
    EXCAT = jnp.concatenate(
        [jnp.concatenate([ex4[:, h, g:g + 1] for g in range(NH)], axis=0)
         for h in range(NH)], axis=1)

    iota_r = jax.lax.broadcasted_iota(jnp.int32, (R, BX), 0)
    OH = (iota_r == zz_ref[0]).astype(jnp.float32)
    den_ref[...] += jnp.dot(OH, EXCAT, preferred_element_type=jnp.float32)
    num_ref[...] += jnp.concatenate(
        [jnp.dot(OH, VRs[h], preferred_element_type=jnp.float32)
         for h in range(NH)], axis=1)


WACC = 128
RACC = 4096
CH = 200
CPA = N // CH
NCHUNK = NH * CPA
SUB = 100


def _attn_sc_body(x_ref, zz_ref, s_ref, wq_ref, bq_ref, wk_ref, bk_ref,
                  wv_ref, bv_ref, vrx_ref):
    qseed = jnp.dot(s_ref[...], wq_ref[...],
                    preferred_element_type=jnp.float32) + bq_ref[...]
    gi = jax.lax.broadcasted_iota(jnp.int32, (H, NH), 0) // DS
    gj = jax.lax.broadcasted_iota(jnp.int32, (H, NH), 1)
    G = (gi == gj).astype(jnp.float32)
    ti = jax.lax.broadcasted_iota(jnp.int32, (NH, H), 1) // DS
    tj = jax.lax.broadcasted_iota(jnp.int32, (NH, H), 0)
    GT = (ti == tj).astype(jnp.float32)

    kfull = jnp.dot(x_ref[...], wk_ref[...],
                    preferred_element_type=jnp.float32) + bk_ref[...]
    vfull = jnp.dot(x_ref[...], wv_ref[...],
                    preferred_element_type=jnp.float32) + bv_ref[...]

    rowmod = jax.lax.broadcasted_iota(jnp.int32, (BX, H), 0) % NH
    QB = jnp.zeros((BX, H), jnp.float32)
    for h in range(NH):
        qtile = jnp.tile(qseed[:, h * DS:(h + 1) * DS], (1, NH))
        QB = jnp.where(rowmod == h, qtile, QB)

    P = jnp.dot(kfull * QB, G, preferred_element_type=jnp.float32)
    exall = jnp.exp(P * (1.0 / 16.0))
    VWr = vfull * jnp.dot(exall, GT, preferred_element_type=jnp.float32)

    vw4 = VWr.reshape(BJ, NH, H)
    ex4 = exall.reshape(BJ, NH, NH)
    pad = jnp.zeros((BX, WACC - DS - 1), jnp.float32)
    for h in range(NH):
        vr = jnp.concatenate([vw4[:, h, g * DS:(g + 1) * DS]
                              for g in range(NH)], axis=0)
        exc = jnp.concatenate([ex4[:, h, g:g + 1] for g in range(NH)],
                              axis=0)
        vrx_ref[h] = jnp.concatenate([vr, exc, pad], axis=1)


def _sc_segsum(vr_hbm, tgt_hbm, zeros_hbm, out_hbm, idx_v, rows_v, acc_sh, sem):
    cid = lax.axis_index("c")
    sid = lax.axis_index("s")
    w = sid * 2 + cid

    @pl.when(sid == 0)
    def _():
        pltpu.sync_copy(zeros_hbm, acc_sh)
    plsc.subcore_barrier()

    NK = NCHUNK // 32 + 1
    for k in range(NK + 1):
        if k < NK:
            buf = k % 2
            q = w + 32 * k

            @pl.when(q < NCHUNK)
            def _():
                g = q // CPA
                c = q % CPA
                pltpu.sync_copy(tgt_hbm.at[g, c], idx_v.at[buf])
                pltpu.async_copy(vr_hbm.at[g, pl.ds(c * CH, CH)],
                                 rows_v.at[buf], sem)
        if k > 0:
            pbuf = (k - 1) % 2
            qp = w + 32 * (k - 1)

            @pl.when(qp < NCHUNK)
            def _():
                gp = qp // CPA
                cp = qp % CPA
                pltpu.make_async_copy(vr_hbm.at[gp, pl.ds(cp * CH, CH)],
                                      rows_v.at[pbuf], sem).wait()
                for b in range(CH // SUB):
                    pltpu.sync_copy(rows_v.at[pbuf, pl.ds(b * SUB, SUB)],
                                    acc_sh.at[idx_v.at[pbuf, b]], add=True)

    plsc.subcore_barrier()

    @pl.when(sid == 0)
    def _():
        pltpu.sync_copy(acc_sh, out_hbm.at[cid])


def _gcn_mat_body(adj_ref, m_ref):
    iota_re = jax.lax.broadcasted_iota(jnp.int32, (R, EC), 0)

    def _edge_chunk(c, M):
        rows = adj_ref[c, 0:1, :]
        cols = adj_ref[c, 1:2, :]
        ohr = (iota_re == rows).astype(jnp.bfloat16)
        ohc = (iota_re == cols).astype(jnp.bfloat16)
        return M + jax.lax.dot_general(
            ohc, ohr, (((1,), (1,)), ((), ())),
            preferred_element_type=jnp.float32)

    M = jax.lax.fori_loop(0, NEC, _edge_chunk,
                          jnp.zeros((R, R), jnp.float32))
    ri = jax.lax.broadcasted_iota(jnp.int32, (R, R), 0)
    rj = jax.lax.broadcasted_iota(jnp.int32, (R, R), 1)
    m_ref[...] = M + (ri == rj).astype(jnp.float32)


def _head_body(acc2_ref, m_ref, s_ref, wq_ref, bq_ref, wo_ref, bo_ref,
               wg_ref, bg_ref, pa_ref, out_ref):
    qseed = jnp.dot(s_ref[...], wq_ref[...],
                    preferred_element_type=jnp.float32) + bq_ref[...]

    a = acc2_ref[0] + acc2_ref[1]
    num = jnp.concatenate([a[h * R:(h + 1) * R, 0:DS] for h in range(NH)],
                          axis=1)
    den4 = jnp.concatenate([a[h * R:(h + 1) * R, DS:DS + 1]
                            for h in range(NH)], axis=1)

    gi = jax.lax.broadcasted_iota(jnp.int32, (NH, H), 1) // DS
    gj = jax.lax.broadcasted_iota(jnp.int32, (NH, H), 0)
    GT = (gi == gj).astype(jnp.float32)
    denR = jnp.dot(den4, GT, preferred_element_type=jnp.float32)

    region = num / (denR + 1e-16)
    O = qseed + region
    O = O + jax.nn.relu(jnp.dot(O, wo_ref[...],
                                preferred_element_type=jnp.float32) + bo_ref[...])
    hW = jnp.dot(O, wg_ref[...], preferred_element_type=jnp.float32)

    M = m_ref[...]
    deg = jnp.sum(M, axis=1, keepdims=True)
    dinv = jax.lax.rsqrt(deg)
    hfin = dinv * jnp.dot(M, dinv * hW, preferred_element_type=jnp.float32)
    hfin = hfin + bg_ref[...]
    a = pa_ref[...]
    hfin = jnp.where(hfin >= 0, hfin, a * hfin)
    hfin = jnp.where(jnp.isnan(hfin), 0.0, hfin)
    hfin = jnp.clip(hfin, -_FMAX, _FMAX)
    out_ref[...] = hfin


def kernel(x, zone, region_adjacency, S, Wq, bq, Wk, bk, Wv, bv, Wo, bo,
           Wg, bg, prelu_a):
    f32 = jnp.float32
    x = x.astype(f32)
    zz = (zone.astype(jnp.int32).reshape(NH, NB, BJ)
          .transpose(1, 0, 2).reshape(NB, 1, BX))
    s2 = S.reshape(1, H).astype(f32)
    bq2 = bq.reshape(1, H).astype(f32)
    bk2 = bk.reshape(1, H).astype(f32)
    bv2 = bv.reshape(1, H).astype(f32)
    bo2 = bo.reshape(1, H).astype(f32)
    bg2 = bg.reshape(1, H).astype(f32)
    adj3 = (region_adjacency.astype(jnp.int32)
            .reshape(2, NEC, EC).transpose(1, 0, 2))
    pa = prelu_a.reshape(1, 1).astype(f32)

    vrx = pl.pallas_call(
        _attn_sc_body,
        grid=(NB,),
        in_specs=[
            pl.BlockSpec((BX, H), lambda i: (i, 0)),
            pl.BlockSpec((1, 1, BX), lambda i: (i, 0, 0)),
            pl.BlockSpec((1, H), lambda i: (0, 0)),
            pl.BlockSpec((H, H), lambda i: (0, 0)),
            pl.BlockSpec((1, H), lambda i: (0, 0)),
            pl.BlockSpec((H, H), lambda i: (0, 0)),
            pl.BlockSpec((1, H), lambda i: (0, 0)),
            pl.BlockSpec((H, H), lambda i: (0, 0)),
            pl.BlockSpec((1, H), lambda i: (0, 0)),
        ],
        out_specs=pl.BlockSpec((NH, BX, WACC), lambda i: (0, i, 0)),
        out_shape=jax.ShapeDtypeStruct((NH, N, WACC), f32),
    )(x, zz, s2, Wq.astype(f32), bq2, Wk.astype(f32), bk2,
      Wv.astype(f32), bv2)

    tgt = (zz.reshape(1, N)
           + R * jnp.arange(NH, dtype=jnp.int32)[:, None]
           ).reshape(NH, CPA, CH // SUB, SUB)
    zeros2 = jnp.zeros((RACC, WACC), f32)
    mesh = plsc.VectorSubcoreMesh(core_axis_name="c", subcore_axis_name="s")
    seg = pl.kernel(
        _sc_segsum,
        mesh=mesh,
        out_type=jax.ShapeDtypeStruct((2, RACC, WACC), f32),
        scratch_types=[
            pltpu.VMEM((2, CH // SUB, SUB), jnp.int32),
            pltpu.VMEM((2, CH, WACC), f32),
            pltpu.VMEM_SHARED((RACC, WACC), f32),
            pltpu.SemaphoreType.DMA,
        ],
    )
    m_mat = pl.pallas_call(
        _gcn_mat_body,
        out_shape=jax.ShapeDtypeStruct((R, R), f32),
    )(adj3)

    acc2 = seg(vrx, tgt, zeros2)

    out = pl.pallas_call(
        _head_body,
        out_shape=jax.ShapeDtypeStruct((R, H), f32),
    )(acc2, m_mat, s2, Wq.astype(f32), bq2, Wo.astype(f32), bo2,
      Wg.astype(f32), bg2, pa)
    return out
